# baseline (speedup 1.0000x reference)
import jax
import jax.numpy as jnp
from jax import lax
from jax.experimental import pallas as pl
from jax.experimental.pallas import tpu as pltpu
from jax.experimental.pallas import tpu_sc as plsc

N = 10000
NP = 10112
E = 320000
H = 128
D = 144
NS = 16
CH = 64
GRP = 10
EPG = GRP * CH
NGRP = E // EPG
STRIPE = NP // NS
K_OUT = (NGRP + NS - 1) // NS


def _stripe_chunks(ch=CH):
    out, off = [], 0
    while off < STRIPE:
        n = min(ch, STRIPE - off)
        out.append((off, n))
        off += n
    return out


def _make_prop(offset_by_core: bool, DD: int = D, DEPTH: int = 4,
               CHv: int = CH, GRPv: int = GRP):
    mesh = plsc.VectorSubcoreMesh(core_axis_name="c", subcore_axis_name="s")
    scratch = [
        pltpu.VMEM_SHARED((NP, DD), jnp.float32),
        pltpu.VMEM((2 * GRPv, CHv), jnp.int32),
        pltpu.VMEM((2 * GRPv, CHv), jnp.int32),
    ] + [pltpu.VMEM((CHv, DD), jnp.float32)] * DEPTH + (
        [pltpu.SemaphoreType.DMA] * (2 * DEPTH + 2))

    def body(table, edges, acc_out, acc_sh, ib0, ib1, *rest):
        ibuf = [ib0, ib1]
        rows = list(rest[:DEPTH])
        gsem = list(rest[DEPTH:2 * DEPTH])
        ssem = list(rest[2 * DEPTH:3 * DEPTH])
        isem = list(rest[3 * DEPTH:3 * DEPTH + 2])
        r0 = rows[0]
        c = lax.axis_index("c")
        s = lax.axis_index("s")
        row0 = s * STRIPE
        zrow = jnp.zeros((16,), jnp.float32)

        def zfill(t, _):
            r0[t // (DD // 16), pl.ds((t % (DD // 16)) * 16, 16)] = zrow
            return 0
        lax.fori_loop(0, CHv * (DD // 16), zfill, 0)
        for off, n in _stripe_chunks(CHv):
            pltpu.sync_copy(r0.at[pl.ds(0, n)],
                            acc_sh.at[pl.ds(row0 + off, n)])
        plsc.subcore_barrier()

        def irow_of(g):
            return (c * NGRP + g) * (2 * GRPv)

        pltpu.sync_copy(edges.at[pl.ds(irow_of(s), 2 * GRPv)], ibuf[0])

        def outer(k, _):
            for p in range(2):
                t = 2 * k + p
                g = s + NS * t

                @pl.when(g < NGRP)
                def _(p=p, g=g):
                    ib = ibuf[p]
                    pb = ibuf[1 - p]
                    base = (p * GRPv) % DEPTH

                    def bi(j):
                        return (base + j) % DEPTH
                    if p == 1:
                        pltpu.make_async_copy(
                            edges.at[pl.ds(irow_of(g), 2 * GRPv)], ib,
                            isem[p]).wait()
                    else:
                        @pl.when(k > 0)
                        def _():
                            pltpu.make_async_copy(
                                edges.at[pl.ds(irow_of(g), 2 * GRPv)], ib,
                                isem[p]).wait()
                    if offset_by_core:
                        off = c * NP
                        for j in range(GRPv):
                            for l in range(CHv // 16):
                                ib[j, pl.ds(l * 16, 16)] = (
                                    ib[j, pl.ds(l * 16, 16)] + off)
                    for j in range(GRPv + 1):
                        if j < GRPv:
                            if j >= DEPTH:
                                pltpu.make_async_copy(
                                    rows[bi(j)],
                                    acc_sh.at[ib.at[GRPv + j - DEPTH]],
                                    ssem[bi(j)]).wait()
                            else:
                                def _pw(j=j):
                                    pltpu.make_async_copy(
                                        rows[bi(j)],
                                        acc_sh.at[pb.at[2 * GRPv + j - DEPTH]],
                                        ssem[bi(j)]).wait()
                                if p == 1:
                                    _pw()
                                else:
                                    pl.when(k > 0)(_pw)
                            pltpu.async_copy(
                                table.at[ib.at[j]], rows[bi(j)], gsem[bi(j)])
                        if j == 4:
                            gn = g + NS

                            @pl.when(gn < NGRP)
                            def _():
                                pltpu.async_copy(
                                    edges.at[pl.ds(irow_of(gn), 2 * GRPv)],
                                    pb, isem[1 - p])
                        if j >= 1:
                            jj = j - 1
                            pltpu.make_async_copy(
                                table.at[ib.at[jj]], rows[bi(jj)],
                                gsem[bi(jj)]).wait()
                            pltpu.async_copy(
                                rows[bi(jj)], acc_sh.at[ib.at[GRPv + jj]],
                                ssem[bi(jj)], add=True)
            return 0
        lax.fori_loop(0, ((NGRP + NS - 1) // NS + 1) // 2, outer, 0)
        for j in range(DEPTH):
            pltpu.make_async_copy(
                rows[j], acc_sh.at[ibuf[0].at[GRPv + j]], ssem[j]).wait()
        plsc.subcore_barrier()

        out0 = c * NP + row0
        for off, n in _stripe_chunks(CHv):
            pltpu.sync_copy(acc_sh.at[pl.ds(row0 + off, n)],
                            r0.at[pl.ds(0, n)])
            pltpu.sync_copy(r0.at[pl.ds(0, n)],
                            acc_out.at[pl.ds(out0 + off, n)])

    return pl.kernel(
        body, mesh=mesh,
        out_type=jax.ShapeDtypeStruct((2 * NP, DD), jnp.float32),
        scratch_types=scratch,
        compiler_params=pltpu.CompilerParams(use_tc_tiling_on_sc=False),
    )


def _make_deg():
    mesh = plsc.VectorSubcoreMesh(core_axis_name="c", subcore_axis_name="s")
    scratch = [
        pltpu.VMEM_SHARED((NP, 16), jnp.float32),
        pltpu.VMEM_SHARED((NP, 16), jnp.float32),
        pltpu.VMEM((2 * GRP, CH), jnp.int32),
        pltpu.VMEM((CH, 16), jnp.float32),
        pltpu.VMEM((CH, 16), jnp.float32),
        pltpu.SemaphoreType.DMA,
        pltpu.SemaphoreType.DMA,
        pltpu.SemaphoreType.DMA,
        pltpu.SemaphoreType.DMA,
        pltpu.SemaphoreType.DMA,
        pltpu.SemaphoreType.DMA,
        pltpu.SemaphoreType.DMA,
        pltpu.SemaphoreType.DMA,
    ]

    def body(edges, dego_out, degi_out, dego_sh, degi_sh, idx, ones16,
             bounce16, s0, s1, s2, s3, s4, s5, s6, s7):
        osem = [s0, s1, s2, s3]
        dsem = [s4, s5, s6, s7]
        c = lax.axis_index("c")
        s = lax.axis_index("s")
        row0 = s * STRIPE
        zrow = jnp.zeros((16,), jnp.float32)

        def zfill16(t, _):
            bounce16[t, :] = zrow
            return 0
        lax.fori_loop(0, CH, zfill16, 0)
        for off, n in _stripe_chunks():
            pltpu.sync_copy(bounce16.at[pl.ds(0, n)],
                            dego_sh.at[pl.ds(row0 + off, n)])
            pltpu.sync_copy(bounce16.at[pl.ds(0, n)],
                            degi_sh.at[pl.ds(row0 + off, n)])
        e0 = jnp.where(lax.iota(jnp.int32, 16) == 0,
                       jnp.float32(1.0), jnp.float32(0.0))

        def ofill(t, _):
            ones16[t, :] = e0
            return 0
        lax.fori_loop(0, CH, ofill, 0)
        plsc.subcore_barrier()

        def grp(k, _):
            g = s + NS * k

            @pl.when(g < NGRP)
            def _():
                irow = (c * NGRP + g) * (2 * GRP)
                pltpu.sync_copy(edges.at[pl.ds(irow, 2 * GRP)], idx)
                for j in range(GRP):
                    if j >= 4:
                        pltpu.make_async_copy(
                            ones16, dego_sh.at[idx.at[j - 4]],
                            osem[j % 4]).wait()
                        pltpu.make_async_copy(
                            ones16, degi_sh.at[idx.at[GRP + j - 4]],
                            dsem[j % 4]).wait()
                    pltpu.async_copy(ones16, dego_sh.at[idx.at[j]],
                                     osem[j % 4], add=True)
                    pltpu.async_copy(ones16, degi_sh.at[idx.at[GRP + j]],
                                     dsem[j % 4], add=True)
                for jj in range(GRP - 4, GRP):
                    pltpu.make_async_copy(
                        ones16, dego_sh.at[idx.at[jj]],
                        osem[jj % 4]).wait()
                    pltpu.make_async_copy(
                        ones16, degi_sh.at[idx.at[GRP + jj]],
                        dsem[jj % 4]).wait()
            return 0
        lax.fori_loop(0, K_OUT, grp, 0)
        plsc.subcore_barrier()

        out0 = c * NP + row0
        for off, n in _stripe_chunks():
            pltpu.sync_copy(dego_sh.at[pl.ds(row0 + off, n)],
                            bounce16.at[pl.ds(0, n)])
            pltpu.sync_copy(bounce16.at[pl.ds(0, n)],
                            dego_out.at[pl.ds(out0 + off, n)])
            pltpu.sync_copy(degi_sh.at[pl.ds(row0 + off, n)],
                            bounce16.at[pl.ds(0, n)])
            pltpu.sync_copy(bounce16.at[pl.ds(0, n)],
                            degi_out.at[pl.ds(out0 + off, n)])

    return pl.kernel(
        body, mesh=mesh,
        out_type=(jax.ShapeDtypeStruct((2 * NP, 16), jnp.float32),
                  jax.ShapeDtypeStruct((2 * NP, 16), jnp.float32)),
        scratch_types=scratch,
        compiler_params=pltpu.CompilerParams(use_tc_tiling_on_sc=False),
    )


_prop1 = _make_prop(offset_by_core=False, DD=H, DEPTH=5)
_prop2 = _make_prop(offset_by_core=True, DD=H, DEPTH=5)
_prop3 = _make_prop(offset_by_core=True, DD=D, DEPTH=8, CHv=32, GRPv=20)
_deg = _make_deg()


_RB_TC = STRIPE


def _stage_a(acc1, hp2, degi):
    return (acc1 + hp2) / (degi[:, 0:1] + 1.0)


def _stage_b(acc2, table2, dego, degi):
    di = degi[:, 0:1]
    z = (acc2 + table2) / (di + 1.0)
    dout = dego[:, 0:1]
    n_out = jnp.where(dout > 0, lax.rsqrt(dout), 0.0)
    n_in = jnp.where(di > 0, lax.rsqrt(di), 0.0)
    table3 = jnp.concatenate(
        [z * n_out, n_out, jnp.zeros((2 * NP, D - H - 1), jnp.float32)],
        axis=1)
    n_in16 = jnp.concatenate(
        [n_in, jnp.zeros((2 * NP, 15), jnp.float32)], axis=1)
    return table3, n_in16


def _ln(x, g, b):
    m = jnp.mean(x, axis=-1, keepdims=True)
    v = jnp.mean((x - m) ** 2, axis=-1, keepdims=True)
    return (x - m) / jnp.sqrt(v + 1e-5) * g + b


def _final_body(h_r, a3g_r, a3a_r, nig_r, nia_r, W1r, b1r, W2r, b2r,
                W3r, b3row_r, fw1_r, fb1_r, fw2_r, fb2_r, lg_r, lb_r, out_r):
    zC = jnp.zeros((H, 16), jnp.float32)
    zc = jnp.zeros((1, 16), jnp.float32)
    blocks = [[], []]
    cvs = [[], []]
    for i in range(8):
        g = 0 if i in (0, 1, 4, 5) else 1
        M = jnp.dot(W1r[i], W2r[i], preferred_element_type=jnp.float32)
        Ci = jnp.dot(M, W3r[i], preferred_element_type=jnp.float32)
        const = jnp.dot(b1r[i:i + 1, :], W2r[i],
                        preferred_element_type=jnp.float32) + b2r[i:i + 1, :]
        ci = jnp.dot(const, W3r[i], preferred_element_type=jnp.float32)
        for gg in range(2):
            blocks[gg].append(Ci if g == gg else zC)
            cvs[gg].append(ci if g == gg else zc)
    Cg = jnp.concatenate(blocks[0], axis=1)
    Ca = jnp.concatenate(blocks[1], axis=1)
    cvg = jnp.concatenate(cvs[0], axis=1)
    cva = jnp.concatenate(cvs[1], axis=1)
    b3row = b3row_r[...]

    h = h_r[...]
    a3g = a3g_r[...]
    a3a = a3a_r[...]
    ni_g = nig_r[...][:, 0:1]
    ni_a = nia_r[...][:, 0:1]
    vg = a3g[:, :H] * ni_g
    va = a3a[:, :H] * ni_a
    sg = a3g[:, H:H + 1] * ni_g
    sa = a3a[:, H:H + 1] * ni_a
    xcat = (jnp.dot(vg, Cg, preferred_element_type=jnp.float32)
            + jnp.dot(va, Ca, preferred_element_type=jnp.float32)
            + sg * cvg + sa * cva + b3row)
    lg = lg_r[...]
    lb = lb_r[...]
    x1 = h + _ln(xcat, lg, lb)
    ffh = jnp.maximum(
        jnp.dot(x1, fw1_r[...], preferred_element_type=jnp.float32)
        + fb1_r[...], 0.0)
    ff = jnp.dot(ffh, fw2_r[...], preferred_element_type=jnp.float32) + fb2_r[...]
    out_r[...] = x1 + _ln(ff, lg, lb)


def _final(hp, acc3, n_in, W1, b1, W2, b2, W3, b3row, fw1, fb1, fw2, fb2,
           lg, lb):
    g = NP // _RB_TC
    whole = lambda shape: pl.BlockSpec(shape, lambda j: tuple(0 for _ in shape))
    return pl.pallas_call(
        _final_body,
        grid=(g,),
        in_specs=[
            pl.BlockSpec((_RB_TC, H), lambda j: (j, 0)),
            pl.BlockSpec((_RB_TC, D), lambda j: (j, 0)),
            pl.BlockSpec((_RB_TC, D), lambda j: (j + NP // _RB_TC, 0)),
            pl.BlockSpec((_RB_TC, 16), lambda j: (j, 0)),
            pl.BlockSpec((_RB_TC, 16), lambda j: (j + NP // _RB_TC, 0)),
            whole((8, H, H)),
            whole((8, H)),
            whole((8, H, H)),
            whole((8, H)),
            whole((8, H, 16)),
            whole((1, H)),
            whole((H, H)),
            whole((1, H)),
            whole((H, H)),
            whole((1, H)),
            whole((1, H)),
            whole((1, H)),
        ],
        out_specs=pl.BlockSpec((_RB_TC, H), lambda j: (j, 0)),
        out_shape=jax.ShapeDtypeStruct((NP, H), jnp.float32),
    )(hp, acc3, acc3, n_in, n_in, W1, b1, W2, b2, W3, b3row,
      fw1, fb1, fw2, fb2, lg, lb)


def _pack_edges(gt_edge_index, attr_edge_index):
    blocks = []
    for ei in (gt_edge_index, attr_edge_index):
        sp = ei[0].reshape(NGRP, GRP, CH)
        dp = ei[1].reshape(NGRP, GRP, CH)
        blocks.append(jnp.concatenate([sp, dp], axis=1))
    return jnp.concatenate(blocks, axis=0).reshape(2 * NGRP * 2 * GRP, CH)


def kernel(h, gt_edge_index, attr_edge_index, qcomp_a, qcomp_b, qcomp_c,
           qcomp_d, W1, b1, W2, b2, W3, b3, ff_W1, ff_b1, ff_W2, ff_b2,
           ln_g, ln_b):
    edges = _pack_edges(gt_edge_index, attr_edge_index)
    hp = jnp.concatenate([h, jnp.zeros((NP - N, H), jnp.float32)], axis=0)
    hp2 = jnp.concatenate([hp, hp], axis=0)
    acc1 = _prop1(hp, edges)
    dego, degi = _deg(edges)
    table2 = _stage_a(acc1, hp2, degi)
    acc2 = _prop2(table2, edges)
    table3, n_in = _stage_b(acc2, table2, dego, degi)
    acc3 = _prop3(table3, edges.reshape(2 * NGRP * 40, 32))
    out = _final(hp, acc3, n_in, W1, b1, W2, b2, W3, b3.reshape(1, H),
                 ff_W1, ff_b1.reshape(1, H), ff_W2, ff_b2.reshape(1, H),
                 ln_g.reshape(1, H), ln_b.reshape(1, H))
    return out[:N]


# speedup vs baseline: 1.0731x; 1.0731x over previous
import jax
import jax.numpy as jnp
from jax import lax
from jax.experimental import pallas as pl
from jax.experimental.pallas import tpu as pltpu
from jax.experimental.pallas import tpu_sc as plsc

N = 10000
NP = 10112
E = 320000
H = 128
D = 144
NS = 16
CH = 64
GRP = 10
EPG = GRP * CH
NGRP = E // EPG
STRIPE = NP // NS
K_OUT = (NGRP + NS - 1) // NS


def _stripe_chunks(ch=CH):
    out, off = [], 0
    while off < STRIPE:
        n = min(ch, STRIPE - off)
        out.append((off, n))
        off += n
    return out


def _make_prop(offset_by_core: bool, DD: int = D, DEPTH: int = 4,
               CHv: int = CH, GRPv: int = GRP):
    mesh = plsc.VectorSubcoreMesh(core_axis_name="c", subcore_axis_name="s")
    scratch = [
        pltpu.VMEM_SHARED((NP, DD), jnp.float32),
        pltpu.VMEM((2 * GRPv, CHv), jnp.int32),
        pltpu.VMEM((2 * GRPv, CHv), jnp.int32),
    ] + [pltpu.VMEM((CHv, DD), jnp.float32)] * DEPTH + (
        [pltpu.SemaphoreType.DMA] * (2 * DEPTH + 2))

    def body(table, edges, acc_out, acc_sh, ib0, ib1, *rest):
        ibuf = [ib0, ib1]
        rows = list(rest[:DEPTH])
        gsem = list(rest[DEPTH:2 * DEPTH])
        ssem = list(rest[2 * DEPTH:3 * DEPTH])
        isem = list(rest[3 * DEPTH:3 * DEPTH + 2])
        r0 = rows[0]
        c = lax.axis_index("c")
        s = lax.axis_index("s")
        row0 = s * STRIPE
        zrow = jnp.zeros((16,), jnp.float32)

        def zfill(t, _):
            r0[t // (DD // 16), pl.ds((t % (DD // 16)) * 16, 16)] = zrow
            return 0
        lax.fori_loop(0, CHv * (DD // 16), zfill, 0)
        for off, n in _stripe_chunks(CHv):
            pltpu.sync_copy(r0.at[pl.ds(0, n)],
                            acc_sh.at[pl.ds(row0 + off, n)])
        plsc.subcore_barrier()

        def irow_of(g):
            return (c * NGRP + g) * (2 * GRPv)

        pltpu.sync_copy(edges.at[pl.ds(irow_of(s), 2 * GRPv)], ibuf[0])

        def outer(k, _):
            for p in range(2):
                t = 2 * k + p
                g = s + NS * t

                @pl.when(g < NGRP)
                def _(p=p, g=g):
                    ib = ibuf[p]
                    pb = ibuf[1 - p]
                    base = (p * GRPv) % DEPTH

                    def bi(j):
                        return (base + j) % DEPTH
                    if p == 1:
                        pltpu.make_async_copy(
                            edges.at[pl.ds(irow_of(g), 2 * GRPv)], ib,
                            isem[p]).wait()
                    else:
                        @pl.when(k > 0)
                        def _():
                            pltpu.make_async_copy(
                                edges.at[pl.ds(irow_of(g), 2 * GRPv)], ib,
                                isem[p]).wait()
                    if offset_by_core:
                        off = c * NP
                        for j in range(GRPv):
                            for l in range(CHv // 16):
                                ib[j, pl.ds(l * 16, 16)] = (
                                    ib[j, pl.ds(l * 16, 16)] + off)
                    for j in range(GRPv + 1):
                        if j < GRPv:
                            if j >= DEPTH:
                                pltpu.make_async_copy(
                                    rows[bi(j)],
                                    acc_sh.at[ib.at[GRPv + j - DEPTH]],
                                    ssem[bi(j)]).wait()
                            else:
                                def _pw(j=j):
                                    pltpu.make_async_copy(
                                        rows[bi(j)],
                                        acc_sh.at[pb.at[2 * GRPv + j - DEPTH]],
                                        ssem[bi(j)]).wait()
                                if p == 1:
                                    _pw()
                                else:
                                    pl.when(k > 0)(_pw)
                            pltpu.async_copy(
                                table.at[ib.at[j]], rows[bi(j)], gsem[bi(j)])
                        if j == 4:
                            gn = g + NS

                            @pl.when(gn < NGRP)
                            def _():
                                pltpu.async_copy(
                                    edges.at[pl.ds(irow_of(gn), 2 * GRPv)],
                                    pb, isem[1 - p])
                        if j >= 1:
                            jj = j - 1
                            pltpu.make_async_copy(
                                table.at[ib.at[jj]], rows[bi(jj)],
                                gsem[bi(jj)]).wait()
                            pltpu.async_copy(
                                rows[bi(jj)], acc_sh.at[ib.at[GRPv + jj]],
                                ssem[bi(jj)], add=True)
            return 0
        lax.fori_loop(0, ((NGRP + NS - 1) // NS + 1) // 2, outer, 0)
        for j in range(DEPTH):
            pltpu.make_async_copy(
                rows[j], acc_sh.at[ibuf[0].at[GRPv + j]], ssem[j]).wait()
        plsc.subcore_barrier()

        out0 = c * NP + row0
        for off, n in _stripe_chunks(CHv):
            pltpu.sync_copy(acc_sh.at[pl.ds(row0 + off, n)],
                            r0.at[pl.ds(0, n)])
            pltpu.sync_copy(r0.at[pl.ds(0, n)],
                            acc_out.at[pl.ds(out0 + off, n)])

    return pl.kernel(
        body, mesh=mesh,
        out_type=jax.ShapeDtypeStruct((2 * NP, DD), jnp.float32),
        scratch_types=scratch,
        compiler_params=pltpu.CompilerParams(use_tc_tiling_on_sc=False),
    )


def _make_deg():
    mesh = plsc.VectorSubcoreMesh(core_axis_name="c", subcore_axis_name="s")
    scratch = [
        pltpu.VMEM_SHARED((NP, 16), jnp.float32),
        pltpu.VMEM_SHARED((NP, 16), jnp.float32),
        pltpu.VMEM((2 * GRP, CH), jnp.int32),
        pltpu.VMEM((CH, 16), jnp.float32),
        pltpu.VMEM((CH, 16), jnp.float32),
        pltpu.SemaphoreType.DMA,
        pltpu.SemaphoreType.DMA,
        pltpu.SemaphoreType.DMA,
        pltpu.SemaphoreType.DMA,
        pltpu.SemaphoreType.DMA,
        pltpu.SemaphoreType.DMA,
        pltpu.SemaphoreType.DMA,
        pltpu.SemaphoreType.DMA,
    ]

    def body(edges, dego_out, degi_out, dego_sh, degi_sh, idx, ones16,
             bounce16, s0, s1, s2, s3, s4, s5, s6, s7):
        osem = [s0, s1, s2, s3]
        dsem = [s4, s5, s6, s7]
        c = lax.axis_index("c")
        s = lax.axis_index("s")
        row0 = s * STRIPE
        zrow = jnp.zeros((16,), jnp.float32)

        def zfill16(t, _):
            bounce16[t, :] = zrow
            return 0
        lax.fori_loop(0, CH, zfill16, 0)
        for off, n in _stripe_chunks():
            pltpu.sync_copy(bounce16.at[pl.ds(0, n)],
                            dego_sh.at[pl.ds(row0 + off, n)])
            pltpu.sync_copy(bounce16.at[pl.ds(0, n)],
                            degi_sh.at[pl.ds(row0 + off, n)])
        e0 = jnp.where(lax.iota(jnp.int32, 16) == 0,
                       jnp.float32(1.0), jnp.float32(0.0))

        def ofill(t, _):
            ones16[t, :] = e0
            return 0
        lax.fori_loop(0, CH, ofill, 0)
        plsc.subcore_barrier()

        def grp(k, _):
            g = s + NS * k

            @pl.when(g < NGRP)
            def _():
                irow = (c * NGRP + g) * (2 * GRP)
                pltpu.sync_copy(edges.at[pl.ds(irow, 2 * GRP)], idx)
                for j in range(GRP):
                    if j >= 4:
                        pltpu.make_async_copy(
                            ones16, dego_sh.at[idx.at[j - 4]],
                            osem[j % 4]).wait()
                        pltpu.make_async_copy(
                            ones16, degi_sh.at[idx.at[GRP + j - 4]],
                            dsem[j % 4]).wait()
                    pltpu.async_copy(ones16, dego_sh.at[idx.at[j]],
                                     osem[j % 4], add=True)
                    pltpu.async_copy(ones16, degi_sh.at[idx.at[GRP + j]],
                                     dsem[j % 4], add=True)
                for jj in range(GRP - 4, GRP):
                    pltpu.make_async_copy(
                        ones16, dego_sh.at[idx.at[jj]],
                        osem[jj % 4]).wait()
                    pltpu.make_async_copy(
                        ones16, degi_sh.at[idx.at[GRP + jj]],
                        dsem[jj % 4]).wait()
            return 0
        lax.fori_loop(0, K_OUT, grp, 0)
        plsc.subcore_barrier()

        out0 = c * NP + row0
        for off, n in _stripe_chunks():
            pltpu.sync_copy(dego_sh.at[pl.ds(row0 + off, n)],
                            bounce16.at[pl.ds(0, n)])
            pltpu.sync_copy(bounce16.at[pl.ds(0, n)],
                            dego_out.at[pl.ds(out0 + off, n)])
            pltpu.sync_copy(degi_sh.at[pl.ds(row0 + off, n)],
                            bounce16.at[pl.ds(0, n)])
            pltpu.sync_copy(bounce16.at[pl.ds(0, n)],
                            degi_out.at[pl.ds(out0 + off, n)])

    return pl.kernel(
        body, mesh=mesh,
        out_type=(jax.ShapeDtypeStruct((2 * NP, 16), jnp.float32),
                  jax.ShapeDtypeStruct((2 * NP, 16), jnp.float32)),
        scratch_types=scratch,
        compiler_params=pltpu.CompilerParams(use_tc_tiling_on_sc=False),
    )


_prop1 = _make_prop(offset_by_core=False, DD=H, DEPTH=5)
_prop2 = _make_prop(offset_by_core=True, DD=H, DEPTH=5)
_prop3 = _make_prop(offset_by_core=True, DD=D, DEPTH=4)
_deg = _make_deg()


_RB_TC = STRIPE


def _stage_a(acc1, hp2, degi):
    return (acc1 + hp2) / (degi[:, 0:1] + 1.0)


def _stage_b(acc2, table2, dego, degi):
    di = degi[:, 0:1]
    z = (acc2 + table2) / (di + 1.0)
    dout = dego[:, 0:1]
    n_out = jnp.where(dout > 0, lax.rsqrt(dout), 0.0)
    n_in = jnp.where(di > 0, lax.rsqrt(di), 0.0)
    table3 = jnp.concatenate(
        [z * n_out, n_out, jnp.zeros((2 * NP, D - H - 1), jnp.float32)],
        axis=1)
    n_in16 = jnp.concatenate(
        [n_in, jnp.zeros((2 * NP, 15), jnp.float32)], axis=1)
    return table3, n_in16


def _ln(x, g, b):
    m = jnp.mean(x, axis=-1, keepdims=True)
    v = jnp.mean((x - m) ** 2, axis=-1, keepdims=True)
    return (x - m) / jnp.sqrt(v + 1e-5) * g + b


def _final_body(h_r, a3g_r, a3a_r, nig_r, nia_r, W1r, b1r, W2r, b2r,
                W3r, b3row_r, fw1_r, fb1_r, fw2_r, fb2_r, lg_r, lb_r, out_r):
    zC = jnp.zeros((H, 16), jnp.float32)
    zc = jnp.zeros((1, 16), jnp.float32)
    blocks = [[], []]
    cvs = [[], []]
    for i in range(8):
        g = 0 if i in (0, 1, 4, 5) else 1
        M = jnp.dot(W1r[i], W2r[i], preferred_element_type=jnp.float32)
        Ci = jnp.dot(M, W3r[i], preferred_element_type=jnp.float32)
        const = jnp.dot(b1r[i:i + 1, :], W2r[i],
                        preferred_element_type=jnp.float32) + b2r[i:i + 1, :]
        ci = jnp.dot(const, W3r[i], preferred_element_type=jnp.float32)
        for gg in range(2):
            blocks[gg].append(Ci if g == gg else zC)
            cvs[gg].append(ci if g == gg else zc)
    Cg = jnp.concatenate(blocks[0], axis=1)
    Ca = jnp.concatenate(blocks[1], axis=1)
    cvg = jnp.concatenate(cvs[0], axis=1)
    cva = jnp.concatenate(cvs[1], axis=1)
    b3row = b3row_r[...]

    h = h_r[...]
    a3g = a3g_r[...]
    a3a = a3a_r[...]
    ni_g = nig_r[...][:, 0:1]
    ni_a = nia_r[...][:, 0:1]
    vg = a3g[:, :H] * ni_g
    va = a3a[:, :H] * ni_a
    sg = a3g[:, H:H + 1] * ni_g
    sa = a3a[:, H:H + 1] * ni_a
    xcat = (jnp.dot(vg, Cg, preferred_element_type=jnp.float32)
            + jnp.dot(va, Ca, preferred_element_type=jnp.float32)
            + sg * cvg + sa * cva + b3row)
    lg = lg_r[...]
    lb = lb_r[...]
    x1 = h + _ln(xcat, lg, lb)
    ffh = jnp.maximum(
        jnp.dot(x1, fw1_r[...], preferred_element_type=jnp.float32)
        + fb1_r[...], 0.0)
    ff = jnp.dot(ffh, fw2_r[...], preferred_element_type=jnp.float32) + fb2_r[...]
    out_r[...] = x1 + _ln(ff, lg, lb)


def _final(hp, acc3, n_in, W1, b1, W2, b2, W3, b3row, fw1, fb1, fw2, fb2,
           lg, lb):
    g = NP // _RB_TC
    whole = lambda shape: pl.BlockSpec(shape, lambda j: tuple(0 for _ in shape))
    return pl.pallas_call(
        _final_body,
        grid=(g,),
        in_specs=[
            pl.BlockSpec((_RB_TC, H), lambda j: (j, 0)),
            pl.BlockSpec((_RB_TC, D), lambda j: (j, 0)),
            pl.BlockSpec((_RB_TC, D), lambda j: (j + NP // _RB_TC, 0)),
            pl.BlockSpec((_RB_TC, 16), lambda j: (j, 0)),
            pl.BlockSpec((_RB_TC, 16), lambda j: (j + NP // _RB_TC, 0)),
            whole((8, H, H)),
            whole((8, H)),
            whole((8, H, H)),
            whole((8, H)),
            whole((8, H, 16)),
            whole((1, H)),
            whole((H, H)),
            whole((1, H)),
            whole((H, H)),
            whole((1, H)),
            whole((1, H)),
            whole((1, H)),
        ],
        out_specs=pl.BlockSpec((_RB_TC, H), lambda j: (j, 0)),
        out_shape=jax.ShapeDtypeStruct((NP, H), jnp.float32),
    )(hp, acc3, acc3, n_in, n_in, W1, b1, W2, b2, W3, b3row,
      fw1, fb1, fw2, fb2, lg, lb)


def _pack_edges(gt_edge_index, attr_edge_index):
    blocks = []
    for ei in (gt_edge_index, attr_edge_index):
        sp = ei[0].reshape(NGRP, GRP, CH)
        dp = ei[1].reshape(NGRP, GRP, CH)
        blocks.append(jnp.concatenate([sp, dp], axis=1))
    return jnp.concatenate(blocks, axis=0).reshape(2 * NGRP * 2 * GRP, CH)


def kernel(h, gt_edge_index, attr_edge_index, qcomp_a, qcomp_b, qcomp_c,
           qcomp_d, W1, b1, W2, b2, W3, b3, ff_W1, ff_b1, ff_W2, ff_b2,
           ln_g, ln_b):
    edges = _pack_edges(gt_edge_index, attr_edge_index)
    hp = jnp.concatenate([h, jnp.zeros((NP - N, H), jnp.float32)], axis=0)
    hp2 = jnp.concatenate([hp, hp], axis=0)
    acc1 = _prop1(hp, edges)
    dego, degi = _deg(edges)
    table2 = _stage_a(acc1, hp2, degi)
    acc2 = _prop2(table2, edges)
    table3, n_in = _stage_b(acc2, table2, dego, degi)
    acc3 = _prop3(table3, edges)
    out = _final(hp, acc3, n_in, W1, b1, W2, b2, W3, b3.reshape(1, H),
                 ff_W1, ff_b1.reshape(1, H), ff_W2, ff_b2.reshape(1, H),
                 ln_g.reshape(1, H), ln_b.reshape(1, H))
    return out[:N]
